# baseline (speedup 1.0000x reference)
import math

import jax
import jax.numpy as jnp
from jax import lax
from jax.experimental import pallas as pl
from jax.experimental.pallas import tpu as pltpu
from jax.experimental.pallas import tpu_sc as plsc

_CUTOFF = 0.5
_E_PARENT = 6_400_000
_E_FILTER = 3_200_000

_K = 5120
_NCHUNKS = _E_FILTER // _K
_NW = 32
_JMAX = -(-_NCHUNKS // _NW)

_S3 = -1.0 / 6.0
_S5 = 1.0 / 120.0
_S7 = -1.0 / 5040.0
_S9 = 1.0 / 362880.0


def _body(vflat_hbm, dist_hbm, idx_hbm, vx_hbm, vy_hbm, vz_hbm,
          distf_hbm, sw_hbm, mask_hbm,
          idx_v, ix0_v, ix1_v, ix2_v, px_v, py_v, pz_v,
          dist_v, sw_v, mask_v, sem_d, sem_v):
    wid = lax.axis_index("s") * 2 + lax.axis_index("c")
    ones_i = jnp.ones((16,), jnp.int32)
    zeros_i = jnp.zeros((16,), jnp.int32)

    def chunk_body(j, carry):
        c = wid + _NW * j

        @pl.when(c < _NCHUNKS)
        def _():
            base = c * _K
            pltpu.sync_copy(idx_hbm.at[pl.ds(base, _K)], idx_v)
            cp_d = pltpu.async_copy(dist_hbm.at[idx_v], dist_v, sem_d)

            def mkidx(i, carry):
                o = i * 16
                t = idx_v[pl.ds(o, 16)] * 3
                ix0_v[pl.ds(o, 16)] = t
                ix1_v[pl.ds(o, 16)] = t + 1
                ix2_v[pl.ds(o, 16)] = t + 2
                return carry

            lax.fori_loop(0, _K // 16, mkidx, 0, unroll=4)

            cp_x = pltpu.async_copy(vflat_hbm.at[ix0_v], px_v, sem_v)
            cp_y = pltpu.async_copy(vflat_hbm.at[ix1_v], py_v, sem_v)
            cp_z = pltpu.async_copy(vflat_hbm.at[ix2_v], pz_v, sem_v)
            cp_d.wait()

            def group(g, carry):
                o = g * 16
                d = dist_v[pl.ds(o, 16)]
                m = d < _CUTOFF
                z = d * (math.pi / _CUTOFF) - (math.pi / 2.0)
                z2 = z * z
                s = z * (1.0 + z2 * (_S3 + z2 * (_S5 + z2 * (_S7 + z2 * _S9))))
                sw_v[pl.ds(o, 16)] = jnp.where(m, 0.5 - 0.5 * s, 0.0)
                mask_v[pl.ds(o, 16)] = jnp.where(m, ones_i, zeros_i)
                return carry

            lax.fori_loop(0, _K // 16, group, 0, unroll=4)

            pltpu.sync_copy(dist_v, distf_hbm.at[pl.ds(base, _K)])
            pltpu.sync_copy(sw_v, sw_hbm.at[pl.ds(base, _K)])
            pltpu.sync_copy(mask_v, mask_hbm.at[pl.ds(base, _K)])
            cp_x.wait()
            cp_y.wait()
            cp_z.wait()
            pltpu.sync_copy(px_v, vx_hbm.at[pl.ds(base, _K)])
            pltpu.sync_copy(py_v, vy_hbm.at[pl.ds(base, _K)])
            pltpu.sync_copy(pz_v, vz_hbm.at[pl.ds(base, _K)])

        return carry

    lax.fori_loop(0, _JMAX, chunk_body, 0)


@jax.jit
def _run(vflat, distances, filter_indices):
    mesh = plsc.VectorSubcoreMesh(core_axis_name="c", subcore_axis_name="s")
    fn = pl.kernel(
        _body,
        out_type=[
            jax.ShapeDtypeStruct((_E_FILTER,), jnp.float32),
            jax.ShapeDtypeStruct((_E_FILTER,), jnp.float32),
            jax.ShapeDtypeStruct((_E_FILTER,), jnp.float32),
            jax.ShapeDtypeStruct((_E_FILTER,), jnp.float32),
            jax.ShapeDtypeStruct((_E_FILTER,), jnp.float32),
            jax.ShapeDtypeStruct((_E_FILTER,), jnp.int32),
        ],
        mesh=mesh,
        scratch_types=[
            pltpu.VMEM((_K,), jnp.int32),
            pltpu.VMEM((_K,), jnp.int32),
            pltpu.VMEM((_K,), jnp.int32),
            pltpu.VMEM((_K,), jnp.int32),
            pltpu.VMEM((_K,), jnp.float32),
            pltpu.VMEM((_K,), jnp.float32),
            pltpu.VMEM((_K,), jnp.float32),
            pltpu.VMEM((_K,), jnp.float32),
            pltpu.VMEM((_K,), jnp.float32),
            pltpu.VMEM((_K,), jnp.int32),
            pltpu.SemaphoreType.DMA,
            pltpu.SemaphoreType.DMA,
        ],
    )
    return fn(vflat, distances, filter_indices)


def kernel(vec, distances, filter_indices):
    vx, vy, vz, dist_f, switch, mask_i32 = _run(
        vec.reshape(-1), distances, filter_indices)
    vec_f = jnp.stack([vx, vy, vz], axis=1)
    return vec_f, dist_f, switch, mask_i32.astype(jnp.bool_)


# speedup vs baseline: 13.3046x; 13.3046x over previous
import math

import jax
import jax.numpy as jnp
from jax import lax
from jax.experimental import pallas as pl
from jax.experimental.pallas import tpu as pltpu
from jax.experimental.pallas import tpu_sc as plsc

_CUTOFF = 0.5
_E_PARENT = 6_400_000
_E_FILTER = 3_200_000

_K = 5120
_NCHUNKS = _E_FILTER // _K
_NW = 32
_JMAX = -(-_NCHUNKS // _NW)

_S3 = -1.0 / 6.0
_S5 = 1.0 / 120.0
_S7 = -1.0 / 5040.0
_S9 = 1.0 / 362880.0


def _body(vx_hbm, vy_hbm, vz_hbm, dist_hbm, idx_hbm,
          ox_hbm, oy_hbm, oz_hbm, distf_hbm, sw_hbm, mask_hbm,
          idx_v, px_v, py_v, pz_v, dist_v, sw_v, mask_v, sem_d, sem_v):
    wid = lax.axis_index("s") * 2 + lax.axis_index("c")
    ones_i = jnp.ones((16,), jnp.int32)
    zeros_i = jnp.zeros((16,), jnp.int32)

    def chunk_body(j, carry):
        c = wid + _NW * j

        @pl.when(c < _NCHUNKS)
        def _():
            base = c * _K
            pltpu.sync_copy(idx_hbm.at[pl.ds(base, _K)], idx_v)
            cp_d = pltpu.async_copy(dist_hbm.at[idx_v], dist_v, sem_d)
            cp_x = pltpu.async_copy(vx_hbm.at[idx_v], px_v, sem_v)
            cp_y = pltpu.async_copy(vy_hbm.at[idx_v], py_v, sem_v)
            cp_z = pltpu.async_copy(vz_hbm.at[idx_v], pz_v, sem_v)
            cp_d.wait()

            def group(g, carry):
                o = g * 16
                d = dist_v[pl.ds(o, 16)]
                m = d < _CUTOFF
                z = d * (math.pi / _CUTOFF) - (math.pi / 2.0)
                z2 = z * z
                s = z * (1.0 + z2 * (_S3 + z2 * (_S5 + z2 * (_S7 + z2 * _S9))))
                sw_v[pl.ds(o, 16)] = jnp.where(m, 0.5 - 0.5 * s, 0.0)
                mask_v[pl.ds(o, 16)] = jnp.where(m, ones_i, zeros_i)
                return carry

            lax.fori_loop(0, _K // 16, group, 0, unroll=4)

            pltpu.sync_copy(dist_v, distf_hbm.at[pl.ds(base, _K)])
            pltpu.sync_copy(sw_v, sw_hbm.at[pl.ds(base, _K)])
            pltpu.sync_copy(mask_v, mask_hbm.at[pl.ds(base, _K)])
            cp_x.wait()
            cp_y.wait()
            cp_z.wait()
            pltpu.sync_copy(px_v, ox_hbm.at[pl.ds(base, _K)])
            pltpu.sync_copy(py_v, oy_hbm.at[pl.ds(base, _K)])
            pltpu.sync_copy(pz_v, oz_hbm.at[pl.ds(base, _K)])

        return carry

    lax.fori_loop(0, _JMAX, chunk_body, 0)


@jax.jit
def _run(vx, vy, vz, distances, filter_indices):
    mesh = plsc.VectorSubcoreMesh(core_axis_name="c", subcore_axis_name="s")
    fn = pl.kernel(
        _body,
        out_type=[
            jax.ShapeDtypeStruct((_E_FILTER,), jnp.float32),
            jax.ShapeDtypeStruct((_E_FILTER,), jnp.float32),
            jax.ShapeDtypeStruct((_E_FILTER,), jnp.float32),
            jax.ShapeDtypeStruct((_E_FILTER,), jnp.float32),
            jax.ShapeDtypeStruct((_E_FILTER,), jnp.float32),
            jax.ShapeDtypeStruct((_E_FILTER,), jnp.int32),
        ],
        mesh=mesh,
        scratch_types=[
            pltpu.VMEM((_K,), jnp.int32),
            pltpu.VMEM((_K,), jnp.float32),
            pltpu.VMEM((_K,), jnp.float32),
            pltpu.VMEM((_K,), jnp.float32),
            pltpu.VMEM((_K,), jnp.float32),
            pltpu.VMEM((_K,), jnp.float32),
            pltpu.VMEM((_K,), jnp.int32),
            pltpu.SemaphoreType.DMA,
            pltpu.SemaphoreType.DMA,
        ],
    )
    return fn(vx, vy, vz, distances, filter_indices)


def kernel(vec, distances, filter_indices):
    ox, oy, oz, dist_f, switch, mask_i32 = _run(
        vec[:, 0], vec[:, 1], vec[:, 2], distances, filter_indices)
    vec_f = jnp.stack([ox, oy, oz], axis=1)
    return vec_f, dist_f, switch, mask_i32.astype(jnp.bool_)
